# baseline (speedup 1.0000x reference)
import jax
import jax.numpy as jnp
from jax import lax
from jax.experimental import pallas as pl
from jax.experimental.pallas import tpu as pltpu
from jax.experimental.pallas import tpu_sc as plsc

B = 16384
D = 64
H1 = 128
H2 = 64
EPS_BN = 1e-5
EPS_LN = 1e-5
TEMP = 0.07

NC = 2
NS = 16
NW = NC * NS
B_PER_W = B // NW
CHUNK = 128
NCHUNK = B_PER_W // CHUNK


def _sc_gather_body(ue_tabT, ie_tabT, uidx, iidx, ue_outT, ie_outT,
                    uidx_v, iidx_v, ucols, icols, sem):
    wid = lax.axis_index("s") * NC + lax.axis_index("c")
    wbase = wid * B_PER_W
    for c in range(NCHUNK):
        pltpu.sync_copy(uidx.at[pl.ds(wbase + c * CHUNK, CHUNK)], uidx_v.at[c])
        pltpu.sync_copy(iidx.at[pl.ds(wbase + c * CHUNK, CHUNK)], iidx_v.at[c])

    @pl.loop(0, D)
    def _feature(d):
        for c in range(NCHUNK):
            cu = pltpu.async_copy(
                ue_tabT.at[d].at[uidx_v.at[c]],
                ucols.at[d, pl.ds(c * CHUNK, CHUNK)], sem)
            ci = pltpu.async_copy(
                ie_tabT.at[d].at[iidx_v.at[c]],
                icols.at[d, pl.ds(c * CHUNK, CHUNK)], sem)
        for c in range(NCHUNK):
            cu.wait()
            ci.wait()

    pltpu.sync_copy(ucols, ue_outT.at[:, pl.ds(wbase, B_PER_W)])
    pltpu.sync_copy(icols, ie_outT.at[:, pl.ds(wbase, B_PER_W)])


def _sc_gather(ue_tabT, ie_tabT, uidx, iidx):
    mesh = plsc.VectorSubcoreMesh(core_axis_name="c", subcore_axis_name="s",
                                  num_cores=NC, num_subcores=NS)
    fn = pl.kernel(
        _sc_gather_body,
        out_type=(jax.ShapeDtypeStruct((D, B), jnp.float32),
                  jax.ShapeDtypeStruct((D, B), jnp.float32)),
        mesh=mesh,
        scratch_types=(
            pltpu.VMEM((NCHUNK, CHUNK), jnp.int32),
            pltpu.VMEM((NCHUNK, CHUNK), jnp.int32),
            pltpu.VMEM((D, B_PER_W), jnp.float32),
            pltpu.VMEM((D, B_PER_W), jnp.float32),
            pltpu.SemaphoreType.DMA,
        ),
        compiler_params=pltpu.CompilerParams(use_tc_tiling_on_sc=False),
    )
    return fn(ue_tabT, ie_tabT, uidx, iidx)


BS = 1024


def _dgT(xT, w):
    return lax.dot_general(xT, w, (((0,), (0,)), ((), ())),
                           preferred_element_type=jnp.float32)


def _tc_body(ufT_ref, ueT_ref, cidx_ref, sidx_ref, ifT_ref, ieT_ref,
             ce_ref, se_ref,
             uW1a_ref, uW1b_ref, uW1c_ref, uW1d_ref, ub1_ref, us1_ref,
             ube1_ref, uW2_ref, ub2_ref, us2_ref, ube2_ref, uW3_ref, ub3_ref,
             iW1a_ref, iW1b_ref, ib1_ref, is1_ref, ibe1_ref,
             iW2_ref, ib2_ref, is2_ref, ibe2_ref, iW3_ref, ib3_ref,
             lng_ref, lnb_ref, out_ref):
    f32 = jnp.float32

    cidx = cidx_ref[...]
    sidx = sidx_ref[...]
    lanes = lax.broadcasted_iota(jnp.int32, (BS, 32), 1)
    onec = (cidx[:, None] == lanes).astype(f32)
    ones = (sidx[:, None] == lanes).astype(f32)
    tc = jnp.dot(ce_ref[...], uW1c_ref[...], preferred_element_type=f32)
    ts = jnp.dot(se_ref[...], uW1d_ref[...], preferred_element_type=f32)

    h = (_dgT(ufT_ref[...], uW1a_ref[...]) + _dgT(ueT_ref[...], uW1b_ref[...])
         + jnp.dot(onec, tc, preferred_element_type=f32)
         + jnp.dot(ones, ts, preferred_element_type=f32) + ub1_ref[...])
    h = jnp.maximum(h, 0.0) * us1_ref[...] + ube1_ref[...]
    h = jnp.maximum(jnp.dot(h, uW2_ref[...], preferred_element_type=f32)
                    + ub2_ref[...], 0.0)
    h = h * us2_ref[...] + ube2_ref[...]
    uo = jnp.dot(h, uW3_ref[...], preferred_element_type=f32) + ub3_ref[...]

    h = (_dgT(ifT_ref[...], iW1a_ref[...]) + _dgT(ieT_ref[...], iW1b_ref[...])
         + ib1_ref[...])
    h = jnp.maximum(h, 0.0) * is1_ref[...] + ibe1_ref[...]
    h = jnp.maximum(jnp.dot(h, iW2_ref[...], preferred_element_type=f32)
                    + ib2_ref[...], 0.0)
    h = h * is2_ref[...] + ibe2_ref[...]
    io = jnp.dot(h, iW3_ref[...], preferred_element_type=f32) + ib3_ref[...]

    def ln(x):
        mu = jnp.mean(x, axis=-1, keepdims=True)
        xc = x - mu
        var = jnp.mean(xc * xc, axis=-1, keepdims=True)
        return xc * lax.rsqrt(var + EPS_LN) * lng_ref[...] + lnb_ref[...]

    uo = ln(uo)
    io = ln(io)
    un = jnp.sum(uo * uo, axis=-1, keepdims=True)
    inn = jnp.sum(io * io, axis=-1, keepdims=True)
    dot = jnp.sum(uo * io, axis=-1, keepdims=True)
    denom = jnp.maximum(jnp.sqrt(un), 1e-12) * jnp.maximum(jnp.sqrt(inn), 1e-12)
    out_ref[...] = dot / denom * (1.0 / TEMP)


def _tc_towers(ufT, ueT, cidx, sidx, ifT, ieT, cep, sep, *weights):
    grid = (B // BS,)
    col = lambda i: (0, i)
    rep2 = lambda i: (0, 0)
    in_specs = [
        pl.BlockSpec((D, BS), col),
        pl.BlockSpec((D, BS), col),
        pl.BlockSpec((BS,), lambda i: (i,)),
        pl.BlockSpec((BS,), lambda i: (i,)),
        pl.BlockSpec((D, BS), col),
        pl.BlockSpec((D, BS), col),
        pl.BlockSpec((32, D), rep2),
        pl.BlockSpec((32, D), rep2),
    ]
    for w in weights:
        in_specs.append(pl.BlockSpec(w.shape, rep2))
    return pl.pallas_call(
        _tc_body,
        grid=grid,
        in_specs=in_specs,
        out_specs=pl.BlockSpec((BS, 1), lambda i: (i, 0)),
        out_shape=jax.ShapeDtypeStruct((B, 1), jnp.float32),
    )(ufT, ueT, cidx, sidx, ifT, ieT, cep, sep, *weights)


def kernel(user_idx, user_features, user_color_idx, user_size_idx, item_idx,
           item_features, ue_tab, ie_tab, ce_tab, se_tab,
           uW1, ub1, ug1, ube1, uW2, ub2, ug2, ube2, uW3, ub3,
           iW1, ib1, ig1, ibe1, iW2, ib2, ig2, ibe2, iW3, ib3,
           ln_g, ln_b):
    f32 = jnp.float32
    uidx = user_idx.astype(jnp.int32)
    iidx = item_idx.astype(jnp.int32)
    cidx = user_color_idx.astype(jnp.int32)
    sidx = user_size_idx.astype(jnp.int32)

    ueT, ieT = _sc_gather(ue_tab.T, ie_tab.T, uidx, iidx)

    bns = 1.0 / jnp.sqrt(jnp.asarray(1.0 + EPS_BN, f32))
    cep = jnp.zeros((32, D), f32).at[:22].set(ce_tab)
    sep = jnp.zeros((32, D), f32).at[:18].set(se_tab)
    row2 = lambda v: v.reshape(1, -1)
    weights = (
        uW1[:, :64].T, uW1[:, 64:128].T, uW1[:, 128:192].T, uW1[:, 192:].T,
        row2(ub1), row2(ug1 * bns), row2(ube1),
        uW2.T, row2(ub2), row2(ug2 * bns), row2(ube2), uW3.T, row2(ub3),
        iW1[:, :64].T, iW1[:, 64:].T,
        row2(ib1), row2(ig1 * bns), row2(ibe1),
        iW2.T, row2(ib2), row2(ig2 * bns), row2(ibe2), iW3.T, row2(ib3),
        row2(ln_g), row2(ln_b))
    return _tc_towers(user_features.T, ueT, cidx, sidx, item_features.T, ieT,
                      cep, sep, *weights)


# speedup vs baseline: 8.7088x; 8.7088x over previous
import jax
import jax.numpy as jnp
from jax import lax
from jax.experimental import pallas as pl
from jax.experimental.pallas import tpu as pltpu
from jax.experimental.pallas import tpu_sc as plsc

B = 16384
D = 64
H1 = 128
H2 = 64
EPS_BN = 1e-5
EPS_LN = 1e-5
TEMP = 0.07

NC = 2
NS = 16
NW = NC * NS
NV = 1000000
TC_W = 128
NTC = (NV + TC_W - 1) // TC_W
TPW = (NTC + NW - 1) // NW
NRING = 4
LANES = 16
CAP = B + LANES
TRASH = CAP
NCH = B // LANES


CHUNK = 128


def _sc_gather_body(ue2_tab, ie2_tab, uidx2, iidx2, ue_out, ie_out,
                    uidx_v, iidx_v, urows, irows, sem):
    wid = lax.axis_index("s") * NC + lax.axis_index("c")
    b_per_w = B // NW
    wbase = wid * b_per_w
    for c in range(b_per_w // CHUNK):
        base = wbase + c * CHUNK
        pltpu.sync_copy(uidx2.at[pl.ds(base, CHUNK)], uidx_v)
        pltpu.sync_copy(iidx2.at[pl.ds(base, CHUNK)], iidx_v)
        cu = pltpu.async_copy(ue2_tab.at[uidx_v], urows, sem)
        ci = pltpu.async_copy(ie2_tab.at[iidx_v], irows, sem)
        cu.wait()
        ci.wait()
        pltpu.sync_copy(urows, ue_out.at[pl.ds(base, CHUNK)])
        pltpu.sync_copy(irows, ie_out.at[pl.ds(base, CHUNK)])


def _sc_gather(ue2_tab, ie2_tab, uidx2, iidx2):
    mesh = plsc.VectorSubcoreMesh(core_axis_name="c", subcore_axis_name="s",
                                  num_cores=NC, num_subcores=NS)
    fn = pl.kernel(
        _sc_gather_body,
        out_type=(jax.ShapeDtypeStruct((B, TC_W), jnp.float32),
                  jax.ShapeDtypeStruct((B, TC_W), jnp.float32)),
        mesh=mesh,
        scratch_types=(
            pltpu.VMEM((CHUNK,), jnp.int32),
            pltpu.VMEM((CHUNK,), jnp.int32),
            pltpu.VMEM((CHUNK, TC_W), jnp.float32),
            pltpu.VMEM((CHUNK, TC_W), jnp.float32),
            pltpu.SemaphoreType.DMA,
        ),
        compiler_params=pltpu.CompilerParams(use_tc_tiling_on_sc=True),
    )
    return fn(ue2_tab, ie2_tab, uidx2, iidx2)


BS = 1024


def _dgT(xT, w):
    return lax.dot_general(xT, w, (((0,), (0,)), ((), ())),
                           preferred_element_type=jnp.float32)


def _tc_body(ufT_ref, ue_ref, upar_ref, cidx_ref, sidx_ref, ifT_ref,
             ie_ref, ipar_ref, ce_ref, se_ref,
             uW1a_ref, uW1b_ref, uW1c_ref, uW1d_ref, ub1_ref, us1_ref,
             ube1_ref, uW2_ref, ub2_ref, us2_ref, ube2_ref, uW3_ref, ub3_ref,
             iW1a_ref, iW1b_ref, ib1_ref, is1_ref, ibe1_ref,
             iW2_ref, ib2_ref, is2_ref, ibe2_ref, iW3_ref, ib3_ref,
             lng_ref, lnb_ref, out_ref):
    f32 = jnp.float32

    cidx = cidx_ref[...]
    sidx = sidx_ref[...]
    lanes = lax.broadcasted_iota(jnp.int32, (BS, 32), 1)
    onec = (cidx[:, None] == lanes).astype(f32)
    ones = (sidx[:, None] == lanes).astype(f32)
    tc = jnp.dot(ce_ref[...], uW1c_ref[...], preferred_element_type=f32)
    ts = jnp.dot(se_ref[...], uW1d_ref[...], preferred_element_type=f32)

    ue = jnp.where(upar_ref[...][:, None] == 1, ue_ref[:, D:], ue_ref[:, :D])
    ie = jnp.where(ipar_ref[...][:, None] == 1, ie_ref[:, D:], ie_ref[:, :D])
    h = (_dgT(ufT_ref[...], uW1a_ref[...])
         + jnp.dot(ue, uW1b_ref[...], preferred_element_type=f32)
         + jnp.dot(onec, tc, preferred_element_type=f32)
         + jnp.dot(ones, ts, preferred_element_type=f32) + ub1_ref[...])
    h = jnp.maximum(h, 0.0) * us1_ref[...] + ube1_ref[...]
    h = jnp.maximum(jnp.dot(h, uW2_ref[...], preferred_element_type=f32)
                    + ub2_ref[...], 0.0)
    h = h * us2_ref[...] + ube2_ref[...]
    uo = jnp.dot(h, uW3_ref[...], preferred_element_type=f32) + ub3_ref[...]

    h = (_dgT(ifT_ref[...], iW1a_ref[...])
         + jnp.dot(ie, iW1b_ref[...], preferred_element_type=f32)
         + ib1_ref[...])
    h = jnp.maximum(h, 0.0) * is1_ref[...] + ibe1_ref[...]
    h = jnp.maximum(jnp.dot(h, iW2_ref[...], preferred_element_type=f32)
                    + ib2_ref[...], 0.0)
    h = h * is2_ref[...] + ibe2_ref[...]
    io = jnp.dot(h, iW3_ref[...], preferred_element_type=f32) + ib3_ref[...]

    def ln(x):
        mu = jnp.mean(x, axis=-1, keepdims=True)
        xc = x - mu
        var = jnp.mean(xc * xc, axis=-1, keepdims=True)
        return xc * lax.rsqrt(var + EPS_LN) * lng_ref[...] + lnb_ref[...]

    uo = ln(uo)
    io = ln(io)
    un = jnp.sum(uo * uo, axis=-1, keepdims=True)
    inn = jnp.sum(io * io, axis=-1, keepdims=True)
    dot = jnp.sum(uo * io, axis=-1, keepdims=True)
    denom = jnp.maximum(jnp.sqrt(un), 1e-12) * jnp.maximum(jnp.sqrt(inn), 1e-12)
    out_ref[...] = dot / denom * (1.0 / TEMP)


def _tc_towers(ufT, ue, upar, cidx, sidx, ifT, ie, ipar, cep, sep, *weights):
    grid = (B // BS,)
    colb = lambda i: (0, i)
    rowb = lambda i: (i, 0)
    rep2 = lambda i: (0, 0)
    in_specs = [
        pl.BlockSpec((D, BS), colb),
        pl.BlockSpec((BS, TC_W), rowb),
        pl.BlockSpec((BS,), lambda i: (i,)),
        pl.BlockSpec((BS,), lambda i: (i,)),
        pl.BlockSpec((BS,), lambda i: (i,)),
        pl.BlockSpec((D, BS), colb),
        pl.BlockSpec((BS, TC_W), rowb),
        pl.BlockSpec((BS,), lambda i: (i,)),
        pl.BlockSpec((32, D), rep2),
        pl.BlockSpec((32, D), rep2),
    ]
    for w in weights:
        in_specs.append(pl.BlockSpec(w.shape, rep2))
    return pl.pallas_call(
        _tc_body,
        grid=grid,
        in_specs=in_specs,
        out_specs=pl.BlockSpec((BS, 1), rowb),
        out_shape=jax.ShapeDtypeStruct((B, 1), jnp.float32),
    )(ufT, ue, upar, cidx, sidx, ifT, ie, ipar, cep, sep, *weights)


def kernel(user_idx, user_features, user_color_idx, user_size_idx, item_idx,
           item_features, ue_tab, ie_tab, ce_tab, se_tab,
           uW1, ub1, ug1, ube1, uW2, ub2, ug2, ube2, uW3, ub3,
           iW1, ib1, ig1, ibe1, iW2, ib2, ig2, ibe2, iW3, ib3,
           ln_g, ln_b):
    f32 = jnp.float32
    uidx = user_idx.astype(jnp.int32)
    iidx = item_idx.astype(jnp.int32)
    cidx = user_color_idx.astype(jnp.int32)
    sidx = user_size_idx.astype(jnp.int32)

    ue, ie = _sc_gather(ue_tab.reshape(NV // 2, 2 * D),
                        ie_tab.reshape(NV // 2, 2 * D),
                        uidx >> 1, iidx >> 1)
    upar = uidx & 1
    ipar = iidx & 1

    bns = 1.0 / jnp.sqrt(jnp.asarray(1.0 + EPS_BN, f32))
    cep = jnp.zeros((32, D), f32).at[:22].set(ce_tab)
    sep = jnp.zeros((32, D), f32).at[:18].set(se_tab)
    row2 = lambda v: v.reshape(1, -1)
    weights = (
        uW1[:, :64].T, uW1[:, 64:128].T, uW1[:, 128:192].T, uW1[:, 192:].T,
        row2(ub1), row2(ug1 * bns), row2(ube1),
        uW2.T, row2(ub2), row2(ug2 * bns), row2(ube2), uW3.T, row2(ub3),
        iW1[:, :64].T, iW1[:, 64:].T,
        row2(ib1), row2(ig1 * bns), row2(ibe1),
        iW2.T, row2(ib2), row2(ig2 * bns), row2(ibe2), iW3.T, row2(ib3),
        row2(ln_g), row2(ln_b))
    return _tc_towers(user_features.T, ue, upar, cidx, sidx,
                      item_features.T, ie, ipar, cep, sep, *weights)


# speedup vs baseline: 8.7710x; 1.0071x over previous
import jax
import jax.numpy as jnp
from jax import lax
from jax.experimental import pallas as pl
from jax.experimental.pallas import tpu as pltpu
from jax.experimental.pallas import tpu_sc as plsc

B = 16384
D = 64
EPS_BN = 1e-5
EPS_LN = 1e-5
TEMP = 0.07

NC = 2
NS = 16
NW = NC * NS
NV = 1000000
TC_W = 128
CHUNK = 128


def _sc_gather_body(ue2_tab, ie2_tab, uidx2, iidx2, ue_out, ie_out,
                    uidx_v, iidx_v, urows, irows, sem):
    wid = lax.axis_index("s") * NC + lax.axis_index("c")
    b_per_w = B // NW
    wbase = wid * b_per_w
    for c in range(b_per_w // CHUNK):
        base = wbase + c * CHUNK
        pltpu.sync_copy(uidx2.at[pl.ds(base, CHUNK)], uidx_v)
        pltpu.sync_copy(iidx2.at[pl.ds(base, CHUNK)], iidx_v)
        cu = pltpu.async_copy(ue2_tab.at[uidx_v], urows, sem)
        ci = pltpu.async_copy(ie2_tab.at[iidx_v], irows, sem)
        cu.wait()
        ci.wait()
        pltpu.sync_copy(urows, ue_out.at[pl.ds(base, CHUNK)])
        pltpu.sync_copy(irows, ie_out.at[pl.ds(base, CHUNK)])


def _sc_gather(ue2_tab, ie2_tab, uidx2, iidx2):
    mesh = plsc.VectorSubcoreMesh(core_axis_name="c", subcore_axis_name="s",
                                  num_cores=NC, num_subcores=NS)
    fn = pl.kernel(
        _sc_gather_body,
        out_type=(jax.ShapeDtypeStruct((B, TC_W), jnp.float32),
                  jax.ShapeDtypeStruct((B, TC_W), jnp.float32)),
        mesh=mesh,
        scratch_types=(
            pltpu.VMEM((CHUNK,), jnp.int32),
            pltpu.VMEM((CHUNK,), jnp.int32),
            pltpu.VMEM((CHUNK, TC_W), jnp.float32),
            pltpu.VMEM((CHUNK, TC_W), jnp.float32),
            pltpu.SemaphoreType.DMA,
        ),
        compiler_params=pltpu.CompilerParams(use_tc_tiling_on_sc=True),
    )
    return fn(ue2_tab, ie2_tab, uidx2, iidx2)


BS = 1024


def _dgT(xT, w):
    return lax.dot_general(xT, w, (((0,), (0,)), ((), ())),
                           preferred_element_type=jnp.float32)


def _tc_body(ufT_ref, ue_ref, upar_ref, cidx_ref, sidx_ref, ifT_ref,
             ie_ref, ipar_ref, ce_ref, se_ref,
             uW1a_ref, uW1b_ref, uW1c_ref, uW1d_ref, ub1_ref, us1_ref,
             ube1_ref, uW2_ref, ub2_ref, us2_ref, ube2_ref, uW3_ref, ub3_ref,
             iW1a_ref, iW1b_ref, ib1_ref, is1_ref, ibe1_ref,
             iW2_ref, ib2_ref, is2_ref, ibe2_ref, iW3_ref, ib3_ref,
             lng_ref, lnb_ref, out_ref):
    f32 = jnp.float32

    cidx = cidx_ref[...]
    sidx = sidx_ref[...]
    lanes = lax.broadcasted_iota(jnp.int32, (BS, 32), 1)
    onec = (cidx[:, None] == lanes).astype(f32)
    ones = (sidx[:, None] == lanes).astype(f32)
    tc = jnp.dot(ce_ref[...], uW1c_ref[...], preferred_element_type=f32)
    ts = jnp.dot(se_ref[...], uW1d_ref[...], preferred_element_type=f32)

    ue = jnp.where(upar_ref[...][:, None] == 1, ue_ref[:, D:], ue_ref[:, :D])
    ie = jnp.where(ipar_ref[...][:, None] == 1, ie_ref[:, D:], ie_ref[:, :D])
    h = (_dgT(ufT_ref[...], uW1a_ref[...])
         + jnp.dot(ue, uW1b_ref[...], preferred_element_type=f32)
         + jnp.dot(onec, tc, preferred_element_type=f32)
         + jnp.dot(ones, ts, preferred_element_type=f32) + ub1_ref[...])
    h = jnp.maximum(h, 0.0) * us1_ref[...] + ube1_ref[...]
    h = jnp.maximum(jnp.dot(h, uW2_ref[...], preferred_element_type=f32)
                    + ub2_ref[...], 0.0)
    h = h * us2_ref[...] + ube2_ref[...]
    uo = jnp.dot(h, uW3_ref[...], preferred_element_type=f32) + ub3_ref[...]

    h = (_dgT(ifT_ref[...], iW1a_ref[...])
         + jnp.dot(ie, iW1b_ref[...], preferred_element_type=f32)
         + ib1_ref[...])
    h = jnp.maximum(h, 0.0) * is1_ref[...] + ibe1_ref[...]
    h = jnp.maximum(jnp.dot(h, iW2_ref[...], preferred_element_type=f32)
                    + ib2_ref[...], 0.0)
    h = h * is2_ref[...] + ibe2_ref[...]
    io = jnp.dot(h, iW3_ref[...], preferred_element_type=f32) + ib3_ref[...]

    def ln(x):
        mu = jnp.mean(x, axis=-1, keepdims=True)
        xc = x - mu
        var = jnp.mean(xc * xc, axis=-1, keepdims=True)
        return xc * lax.rsqrt(var + EPS_LN) * lng_ref[...] + lnb_ref[...]

    uo = ln(uo)
    io = ln(io)
    un = jnp.sum(uo * uo, axis=-1, keepdims=True)
    inn = jnp.sum(io * io, axis=-1, keepdims=True)
    dot = jnp.sum(uo * io, axis=-1, keepdims=True)
    denom = jnp.maximum(jnp.sqrt(un), 1e-12) * jnp.maximum(jnp.sqrt(inn), 1e-12)
    out_ref[...] = dot / denom * (1.0 / TEMP)


def _tc_towers(ufT, ue, upar, cidx, sidx, ifT, ie, ipar, cep, sep, *weights):
    grid = (B // BS,)
    colb = lambda i: (0, i)
    rowb = lambda i: (i, 0)
    rep2 = lambda i: (0, 0)
    in_specs = [
        pl.BlockSpec((D, BS), colb),
        pl.BlockSpec((BS, TC_W), rowb),
        pl.BlockSpec((BS,), lambda i: (i,)),
        pl.BlockSpec((BS,), lambda i: (i,)),
        pl.BlockSpec((BS,), lambda i: (i,)),
        pl.BlockSpec((D, BS), colb),
        pl.BlockSpec((BS, TC_W), rowb),
        pl.BlockSpec((BS,), lambda i: (i,)),
        pl.BlockSpec((32, D), rep2),
        pl.BlockSpec((32, D), rep2),
    ]
    for w in weights:
        in_specs.append(pl.BlockSpec(w.shape, rep2))
    return pl.pallas_call(
        _tc_body,
        grid=grid,
        in_specs=in_specs,
        out_specs=pl.BlockSpec((BS, 1), rowb),
        out_shape=jax.ShapeDtypeStruct((B, 1), jnp.float32),
    )(ufT, ue, upar, cidx, sidx, ifT, ie, ipar, cep, sep, *weights)


def kernel(user_idx, user_features, user_color_idx, user_size_idx, item_idx,
           item_features, ue_tab, ie_tab, ce_tab, se_tab,
           uW1, ub1, ug1, ube1, uW2, ub2, ug2, ube2, uW3, ub3,
           iW1, ib1, ig1, ibe1, iW2, ib2, ig2, ibe2, iW3, ib3,
           ln_g, ln_b):
    f32 = jnp.float32
    uidx = user_idx.astype(jnp.int32)
    iidx = item_idx.astype(jnp.int32)
    cidx = user_color_idx.astype(jnp.int32)
    sidx = user_size_idx.astype(jnp.int32)

    ue, ie = _sc_gather(ue_tab.reshape(NV // 2, 2 * D),
                        ie_tab.reshape(NV // 2, 2 * D),
                        uidx >> 1, iidx >> 1)
    upar = uidx & 1
    ipar = iidx & 1

    bns = 1.0 / jnp.sqrt(jnp.asarray(1.0 + EPS_BN, f32))
    cep = jnp.zeros((32, D), f32).at[:22].set(ce_tab)
    sep = jnp.zeros((32, D), f32).at[:18].set(se_tab)
    row2 = lambda v: v.reshape(1, -1)
    weights = (
        uW1[:, :64].T, uW1[:, 64:128].T, uW1[:, 128:192].T, uW1[:, 192:].T,
        row2(ub1), row2(ug1 * bns), row2(ube1),
        uW2.T, row2(ub2), row2(ug2 * bns), row2(ube2), uW3.T, row2(ub3),
        iW1[:, :64].T, iW1[:, 64:].T,
        row2(ib1), row2(ig1 * bns), row2(ibe1),
        iW2.T, row2(ib2), row2(ig2 * bns), row2(ibe2), iW3.T, row2(ib3),
        row2(ln_g), row2(ln_b))
    return _tc_towers(user_features.T, ue, upar, cidx, sidx,
                      item_features.T, ie, ipar, cep, sep, *weights)


# speedup vs baseline: 19.1013x; 2.1778x over previous
import jax
import jax.numpy as jnp
from jax import lax
from jax.experimental import pallas as pl
from jax.experimental.pallas import tpu as pltpu
from jax.experimental.pallas import tpu_sc as plsc

B = 16384
D = 64
EPS_BN = 1e-5
EPS_LN = 1e-5
TEMP = 0.07

NC = 2
NS = 16
NW = NC * NS
NV = 1000000
TC_W = 128
CHUNK = 128


def _sc_gather_body(ue2_tab, ie2_tab, uidx2, iidx2, ue_out, ie_out,
                    uidx_v, iidx_v, urows, irows, sem):
    wid = lax.axis_index("s") * NC + lax.axis_index("c")
    b_per_w = B // NW
    wbase = wid * b_per_w
    for c in range(b_per_w // CHUNK):
        base = wbase + c * CHUNK
        pltpu.sync_copy(uidx2.at[pl.ds(base, CHUNK)], uidx_v)
        pltpu.sync_copy(iidx2.at[pl.ds(base, CHUNK)], iidx_v)
        cu = pltpu.async_copy(ue2_tab.at[uidx_v], urows, sem)
        ci = pltpu.async_copy(ie2_tab.at[iidx_v], irows, sem)
        cu.wait()
        ci.wait()
        pltpu.sync_copy(urows, ue_out.at[pl.ds(base, CHUNK)])
        pltpu.sync_copy(irows, ie_out.at[pl.ds(base, CHUNK)])


def _sc_gather(ue2_tab, ie2_tab, uidx2, iidx2):
    mesh = plsc.VectorSubcoreMesh(core_axis_name="c", subcore_axis_name="s",
                                  num_cores=NC, num_subcores=NS)
    fn = pl.kernel(
        _sc_gather_body,
        out_type=(jax.ShapeDtypeStruct((B, TC_W), jnp.float32),
                  jax.ShapeDtypeStruct((B, TC_W), jnp.float32)),
        mesh=mesh,
        scratch_types=(
            pltpu.VMEM((CHUNK,), jnp.int32),
            pltpu.VMEM((CHUNK,), jnp.int32),
            pltpu.VMEM((CHUNK, TC_W), jnp.float32),
            pltpu.VMEM((CHUNK, TC_W), jnp.float32),
            pltpu.SemaphoreType.DMA,
        ),
        compiler_params=pltpu.CompilerParams(use_tc_tiling_on_sc=True),
    )
    return fn(ue2_tab, ie2_tab, uidx2, iidx2)


RB = 8192


NPAIR = (NV + 2 * RB - 1) // (2 * RB)
NPACK = NPAIR * RB


def _repack_body(lo_ref, hi_ref, out_ref):
    out_ref[...] = jnp.concatenate(
        [jnp.transpose(lo_ref[...]), jnp.transpose(hi_ref[...])], axis=1)


def _tc_repack(tabT):
    return pl.pallas_call(
        _repack_body,
        grid=(NPAIR,),
        in_specs=[
            pl.BlockSpec((D, RB), lambda i: (0, 2 * i)),
            pl.BlockSpec((D, RB),
                         lambda i: (0, jnp.minimum(2 * i + 1, NV // RB))),
        ],
        out_specs=pl.BlockSpec((RB, 2 * D), lambda i: (i, 0)),
        out_shape=jax.ShapeDtypeStruct((NPACK, 2 * D), jnp.float32),
    )(tabT, tabT)


BS = 1024


def _dgT(xT, w):
    return lax.dot_general(xT, w, (((0,), (0,)), ((), ())),
                           preferred_element_type=jnp.float32)


def _tc_body(ufT_ref, ue_ref, upar_ref, cidx_ref, sidx_ref, ifT_ref,
             ie_ref, ipar_ref, ce_ref, se_ref,
             uW1a_ref, uW1b_ref, uW1c_ref, uW1d_ref, ub1_ref, us1_ref,
             ube1_ref, uW2_ref, ub2_ref, us2_ref, ube2_ref, uW3_ref, ub3_ref,
             iW1a_ref, iW1b_ref, ib1_ref, is1_ref, ibe1_ref,
             iW2_ref, ib2_ref, is2_ref, ibe2_ref, iW3_ref, ib3_ref,
             lng_ref, lnb_ref, out_ref):
    f32 = jnp.float32

    cidx = cidx_ref[...]
    sidx = sidx_ref[...]
    lanes = lax.broadcasted_iota(jnp.int32, (BS, 32), 1)
    onec = (cidx[:, None] == lanes).astype(f32)
    ones = (sidx[:, None] == lanes).astype(f32)
    tc = jnp.dot(ce_ref[...], uW1c_ref[...], preferred_element_type=f32)
    ts = jnp.dot(se_ref[...], uW1d_ref[...], preferred_element_type=f32)

    ue = jnp.where(upar_ref[...][:, None] == 1, ue_ref[:, D:], ue_ref[:, :D])
    ie = jnp.where(ipar_ref[...][:, None] == 1, ie_ref[:, D:], ie_ref[:, :D])
    h = (_dgT(ufT_ref[...], uW1a_ref[...])
         + jnp.dot(ue, uW1b_ref[...], preferred_element_type=f32)
         + jnp.dot(onec, tc, preferred_element_type=f32)
         + jnp.dot(ones, ts, preferred_element_type=f32) + ub1_ref[...])
    h = jnp.maximum(h, 0.0) * us1_ref[...] + ube1_ref[...]
    h = jnp.maximum(jnp.dot(h, uW2_ref[...], preferred_element_type=f32)
                    + ub2_ref[...], 0.0)
    h = h * us2_ref[...] + ube2_ref[...]
    uo = jnp.dot(h, uW3_ref[...], preferred_element_type=f32) + ub3_ref[...]

    h = (_dgT(ifT_ref[...], iW1a_ref[...])
         + jnp.dot(ie, iW1b_ref[...], preferred_element_type=f32)
         + ib1_ref[...])
    h = jnp.maximum(h, 0.0) * is1_ref[...] + ibe1_ref[...]
    h = jnp.maximum(jnp.dot(h, iW2_ref[...], preferred_element_type=f32)
                    + ib2_ref[...], 0.0)
    h = h * is2_ref[...] + ibe2_ref[...]
    io = jnp.dot(h, iW3_ref[...], preferred_element_type=f32) + ib3_ref[...]

    def ln(x):
        mu = jnp.mean(x, axis=-1, keepdims=True)
        xc = x - mu
        var = jnp.mean(xc * xc, axis=-1, keepdims=True)
        return xc * lax.rsqrt(var + EPS_LN) * lng_ref[...] + lnb_ref[...]

    uo = ln(uo)
    io = ln(io)
    un = jnp.sum(uo * uo, axis=-1, keepdims=True)
    inn = jnp.sum(io * io, axis=-1, keepdims=True)
    dot = jnp.sum(uo * io, axis=-1, keepdims=True)
    denom = jnp.maximum(jnp.sqrt(un), 1e-12) * jnp.maximum(jnp.sqrt(inn), 1e-12)
    out_ref[...] = dot / denom * (1.0 / TEMP)


def _tc_towers(ufT, ue, upar, cidx, sidx, ifT, ie, ipar, cep, sep, *weights):
    grid = (B // BS,)
    colb = lambda i: (0, i)
    rowb = lambda i: (i, 0)
    rep2 = lambda i: (0, 0)
    in_specs = [
        pl.BlockSpec((D, BS), colb),
        pl.BlockSpec((BS, TC_W), rowb),
        pl.BlockSpec((BS,), lambda i: (i,)),
        pl.BlockSpec((BS,), lambda i: (i,)),
        pl.BlockSpec((BS,), lambda i: (i,)),
        pl.BlockSpec((D, BS), colb),
        pl.BlockSpec((BS, TC_W), rowb),
        pl.BlockSpec((BS,), lambda i: (i,)),
        pl.BlockSpec((32, D), rep2),
        pl.BlockSpec((32, D), rep2),
    ]
    for w in weights:
        in_specs.append(pl.BlockSpec(w.shape, rep2))
    return pl.pallas_call(
        _tc_body,
        grid=grid,
        in_specs=in_specs,
        out_specs=pl.BlockSpec((BS, 1), rowb),
        out_shape=jax.ShapeDtypeStruct((B, 1), jnp.float32),
    )(ufT, ue, upar, cidx, sidx, ifT, ie, ipar, cep, sep, *weights)


def kernel(user_idx, user_features, user_color_idx, user_size_idx, item_idx,
           item_features, ue_tab, ie_tab, ce_tab, se_tab,
           uW1, ub1, ug1, ube1, uW2, ub2, ug2, ube2, uW3, ub3,
           iW1, ib1, ig1, ibe1, iW2, ib2, ig2, ibe2, iW3, ib3,
           ln_g, ln_b):
    f32 = jnp.float32
    uidx = user_idx.astype(jnp.int32)
    iidx = item_idx.astype(jnp.int32)
    cidx = user_color_idx.astype(jnp.int32)
    sidx = user_size_idx.astype(jnp.int32)

    uq, ur = uidx >> 13, uidx & (RB - 1)
    iq, ir = iidx >> 13, iidx & (RB - 1)
    ue, ie = _sc_gather(_tc_repack(ue_tab.T), _tc_repack(ie_tab.T),
                        ((uq >> 1) << 13) | ur, ((iq >> 1) << 13) | ir)
    upar = uq & 1
    ipar = iq & 1

    bns = 1.0 / jnp.sqrt(jnp.asarray(1.0 + EPS_BN, f32))
    cep = jnp.zeros((32, D), f32).at[:22].set(ce_tab)
    sep = jnp.zeros((32, D), f32).at[:18].set(se_tab)
    row2 = lambda v: v.reshape(1, -1)
    weights = (
        uW1[:, :64].T, uW1[:, 64:128].T, uW1[:, 128:192].T, uW1[:, 192:].T,
        row2(ub1), row2(ug1 * bns), row2(ube1),
        uW2.T, row2(ub2), row2(ug2 * bns), row2(ube2), uW3.T, row2(ub3),
        iW1[:, :64].T, iW1[:, 64:].T,
        row2(ib1), row2(ig1 * bns), row2(ibe1),
        iW2.T, row2(ib2), row2(ig2 * bns), row2(ibe2), iW3.T, row2(ib3),
        row2(ln_g), row2(ln_b))
    return _tc_towers(user_features.T, ue, upar, cidx, sidx,
                      item_features.T, ie, ipar, cep, sep, *weights)
